# Initial kernel scaffold; baseline (speedup 1.0000x reference)
import jax
import jax.numpy as jnp
from jax.experimental import pallas as pl


def kernel(x, boxes):
    raise NotImplementedError("write your pallas kernel here")


# speedup vs baseline: 1.5633x; 1.5633x over previous
import functools

import jax
import jax.numpy as jnp
import numpy as np
from jax import lax
from jax.experimental import pallas as pl
from jax.experimental.pallas import tpu as pltpu
from jax.experimental.pallas import tpu_sc as plsc

_KH, _KW = 7, 7
_NBINS = _KH * _KW
_NLANE = 16
_NGRP = 4
_H = _W = 128
_C = 256
_N = 5000
_NWORK = 32
_BPW = 160
_NPAD = _NWORK * _BPW


def _body(feat, boxes_f, out, bxv, ib0, ib1, ib2, ib3, rb0, rb1, rb2, rb3,
          ob, gsem):
    wid = lax.axis_index("s") * 2 + lax.axis_index("c")
    start = wid * _BPW
    pltpu.sync_copy(boxes_f.at[pl.ds(start * 4, _BPW * 4)], bxv)

    lane = jnp.arange(_NLANE, dtype=jnp.int32)

    def box_body(b, _):
        @pl.when(start + b < _N)
        def _():
            bsplat = jnp.full((_NLANE,), 4 * b, jnp.int32)
            r0 = plsc.load_gather(bxv, [bsplat])
            r1 = plsc.load_gather(bxv, [bsplat + 1])
            r2 = plsc.load_gather(bxv, [bsplat + 2])
            r3 = plsc.load_gather(bxv, [bsplat + 3])
            ymin = jnp.minimum(r0, r2)
            ymax = jnp.maximum(r0, r2)
            xmin = jnp.minimum(r1, r3)
            xmax = jnp.maximum(r1, r3)

            wv = []
            for g in range(_NGRP):
                binv = lane + (g * _NLANE)
                ky = (binv * 37) >> 8
                kx = binv - ky * _KW
                gy = (ky.astype(jnp.float32) + 0.5) / np.float32(_KH)
                gx = (kx.astype(jnp.float32) + 0.5) / np.float32(_KW)
                py = (ymin + (ymax - ymin) * gy) * np.float32(_H - 1)
                px = (xmin + (xmax - xmin) * gx) * np.float32(_W - 1)
                y0f = py.astype(jnp.int32)
                x0f = px.astype(jnp.int32)
                wy = py - y0f.astype(jnp.float32)
                wx = px - x0f.astype(jnp.float32)
                y0 = jnp.minimum(y0f, _H - 1)
                x0 = jnp.minimum(x0f, _W - 1)
                y1c = jnp.minimum(y0 + 1, _H - 1)
                x1c = jnp.minimum(x0 + 1, _W - 1)
                i00 = y0 * _W + x0
                i01 = y0 * _W + x1c
                i10 = y1c * _W + x0
                i11 = y1c * _W + x1c
                sl = pl.ds(g * _NLANE, _NLANE)
                ib0[sl] = i00
                ib1[sl] = i01
                ib2[sl] = i10
                ib3[sl] = i11
                omy = 1.0 - wy
                omx = 1.0 - wx
                wv.append((omy * omx, omy * wx, wy * omx, wy * wx))

            ng = 56
            cps = [
                pltpu.async_copy(feat.at[ib.at[pl.ds(0, ng)]],
                                 rb.at[pl.ds(0, ng)], gsem)
                for ib, rb in ((ib0, rb0), (ib1, rb1), (ib2, rb2), (ib3, rb3))
            ]
            for cp in cps:
                cp.wait()

            for g in range(_NGRP):
                roww = lane + (g * _NLANE)
                w00, w01, w10, w11 = wv[g]

                obase = roww * _C

                def c_body(c, _, roww=roww, obase=obase, w00=w00, w01=w01,
                           w10=w10, w11=w11):
                    colv = jnp.full((_NLANE,), c, jnp.int32)
                    f00 = plsc.load_gather(rb0, [roww, colv])
                    f01 = plsc.load_gather(rb1, [roww, colv])
                    f10 = plsc.load_gather(rb2, [roww, colv])
                    f11 = plsc.load_gather(rb3, [roww, colv])
                    acc = w00 * f00 + w01 * f01 + w10 * f10 + w11 * f11
                    plsc.store_scatter(ob, [obase + colv], acc)
                    return 0

                lax.fori_loop(0, _C, c_body, 0)

            pltpu.sync_copy(ob.at[pl.ds(0, _NBINS * _C)],
                            out.at[pl.ds((start + b) * (_NBINS * _C),
                                         _NBINS * _C)])

        return 0

    lax.fori_loop(0, _BPW, box_body, 0)


@jax.jit
def _roialign_sc(feat, boxes_f):
    kfn = pl.kernel(
        _body,
        out_type=jax.ShapeDtypeStruct((_N * _NBINS * _C,), jnp.float32),
        mesh=plsc.VectorSubcoreMesh(core_axis_name="c", subcore_axis_name="s"),
        scratch_types=[
            pltpu.VMEM((_BPW * 4,), jnp.float32),
            pltpu.VMEM((4 * _NLANE,), jnp.int32),
            pltpu.VMEM((4 * _NLANE,), jnp.int32),
            pltpu.VMEM((4 * _NLANE,), jnp.int32),
            pltpu.VMEM((4 * _NLANE,), jnp.int32),
            pltpu.VMEM((_NGRP * _NLANE, _C), jnp.float32),
            pltpu.VMEM((_NGRP * _NLANE, _C), jnp.float32),
            pltpu.VMEM((_NGRP * _NLANE, _C), jnp.float32),
            pltpu.VMEM((_NGRP * _NLANE, _C), jnp.float32),
            pltpu.VMEM((_NGRP * _NLANE * _C,), jnp.float32),
            pltpu.SemaphoreType.DMA,
        ],
        compiler_params=pltpu.CompilerParams(needs_layout_passes=False),
    )
    return kfn(feat, boxes_f)


def kernel(x, boxes):
    feat = x[0].reshape(_H * _W, _C)
    boxes_f = jnp.pad(boxes, ((0, _NPAD - _N), (0, 0))).reshape(-1)
    out = _roialign_sc(feat, jnp.asarray(boxes_f, jnp.float32))
    return out.reshape(_N, _KH, _KW, _C)
